# Initial kernel scaffold; baseline (speedup 1.0000x reference)
import jax
import jax.numpy as jnp
from jax.experimental import pallas as pl


def kernel(x, edge_index, batch, W1_0, b1_0, gamma_0, beta_0, rm_0, rv_0, W2_0, b2_0, eps_0, W1_1, b1_1, gamma_1, beta_1, rm_1, rv_1, W2_1, b2_1, eps_1, W1_2, b1_2, gamma_2, beta_2, rm_2, rv_2, W2_2, b2_2, eps_2, lin1_W, lin1_b, lin2_W, lin2_b):
    raise NotImplementedError("write your pallas kernel here")


# speedup vs baseline: 9.8732x; 9.8732x over previous
import functools

import jax
import jax.numpy as jnp
from jax import lax
from jax.experimental import pallas as pl
from jax.experimental.pallas import tpu as pltpu
from jax.experimental.pallas import tpu_sc as plsc

_N, _E, _D, _H, _O, _G, _L = 10000, 320000, 128, 128, 128, 64, 3

_NC, _NS = 2, 16
_NW = _NC * _NS
_CH = 128
_CPT = 80
_EPAD = _NW * _CPT * _CH
_NCHUNK = _EPAD // _CH
_TRASH = 112
_NACC = _N + _TRASH
_RPT = _NACC // _NS
_HCPT = _CPT // 2

_R = 1000
_NB = _N // _R

_HIGH = lax.Precision.HIGHEST


def _sc_agg_body(h_hbm, src_hbm, dst_hbm, zeros_hbm, out_hbm,
                 srcv, dstv, rows0, rows1, acc, g0, g1):
    c = lax.axis_index("c")
    s = lax.axis_index("s")
    tile = c * _NS + s
    r0 = s * _RPT

    pltpu.sync_copy(zeros_hbm, acc.at[pl.ds(r0, _RPT)])

    plsc.subcore_barrier()

    for half in range(2):
        ch0 = tile * _CPT + half * _HCPT
        pltpu.sync_copy(src_hbm.at[pl.ds(ch0, _HCPT)], srcv)
        pltpu.sync_copy(dst_hbm.at[pl.ds(ch0, _HCPT)], dstv)
        pltpu.async_copy(h_hbm.at[srcv.at[0]], rows0, g0)

        def step(jj, carry):
            pltpu.async_copy(h_hbm.at[srcv.at[jj + 1]], rows1, g1)
            pltpu.make_async_copy(h_hbm.at[srcv.at[jj]], rows0, g0).wait()
            pltpu.sync_copy(rows0, acc.at[dstv.at[jj]], add=True)

            @pl.when(jj + 2 < _HCPT)
            def _():
                pltpu.async_copy(h_hbm.at[srcv.at[jj + 2]], rows0, g0)

            pltpu.make_async_copy(h_hbm.at[srcv.at[jj + 1]], rows1, g1).wait()
            pltpu.sync_copy(rows1, acc.at[dstv.at[jj + 1]], add=True)
            return carry

        lax.fori_loop(0, _HCPT // 2, lambda k, cr: step(2 * k, cr), 0)

    plsc.subcore_barrier()
    pltpu.sync_copy(acc.at[pl.ds(r0, _RPT)], out_hbm.at[c, pl.ds(r0, _RPT)])


_sc_agg = functools.partial(
    pl.kernel,
    out_type=jax.ShapeDtypeStruct((_NC, _NACC, _H), jnp.float32),
    mesh=plsc.VectorSubcoreMesh(core_axis_name="c", subcore_axis_name="s",
                                num_cores=_NC, num_subcores=_NS),
    scratch_types=[
        pltpu.VMEM((_HCPT, _CH), jnp.int32),
        pltpu.VMEM((_HCPT, _CH), jnp.int32),
        pltpu.VMEM((_CH, _H), jnp.float32),
        pltpu.VMEM((_CH, _H), jnp.float32),
        pltpu.VMEM_SHARED((_NACC, _H), jnp.float32),
        pltpu.SemaphoreType.DMA,
        pltpu.SemaphoreType.DMA,
    ],
)(_sc_agg_body)


def _mlp_body_common(seps_ref, batch_ref, h_ref, p0_ref, p1_ref,
                     W1_ref, b1_ref, scale_ref, shift_ref, W2_ref, b2_ref,
                     hout_ref, pooled_ref):
    i = pl.program_id(0)
    h = h_ref[...]
    agg = p0_ref[0] + p1_ref[0]
    t = h * seps_ref[0, 0] + agg
    t = jnp.maximum(jnp.dot(t, W1_ref[...], precision=_HIGH,
                            preferred_element_type=jnp.float32) + b1_ref[...], 0.0)
    t = jnp.maximum(t * scale_ref[...] + shift_ref[...], 0.0)
    t = jnp.dot(t, W2_ref[...], precision=_HIGH,
                preferred_element_type=jnp.float32) + b2_ref[...]
    t = jnp.maximum(t, 0.0)
    hout_ref[...] = t

    bb = batch_ref[0, 0, :]
    seg = lax.broadcasted_iota(jnp.int32, (_G, _R), 0)
    oht = (seg == jnp.reshape(bb, (1, _R))).astype(jnp.float32)
    contrib = jnp.dot(oht, t, precision=_HIGH,
                      preferred_element_type=jnp.float32)

    @pl.when(i == 0)
    def _():
        pooled_ref[...] = contrib

    @pl.when(i != 0)
    def _():
        pooled_ref[...] += contrib

    return i


def _mlp_body(seps_ref, batch_ref, h_ref, p0_ref, p1_ref,
              W1_ref, b1_ref, scale_ref, shift_ref, W2_ref, b2_ref,
              hout_ref, pooled_ref):
    _mlp_body_common(seps_ref, batch_ref, h_ref, p0_ref, p1_ref,
                     W1_ref, b1_ref, scale_ref, shift_ref, W2_ref, b2_ref,
                     hout_ref, pooled_ref)


def _mlp_head_body(seps_ref, batch_ref, h_ref, p0_ref, p1_ref,
                   W1_ref, b1_ref, scale_ref, shift_ref, W2_ref, b2_ref,
                   l1W_ref, l1b_ref, l2W_ref, l2b_ref,
                   hout_ref, pooled_ref, out_ref):
    i = _mlp_body_common(seps_ref, batch_ref, h_ref, p0_ref, p1_ref,
                         W1_ref, b1_ref, scale_ref, shift_ref, W2_ref, b2_ref,
                         hout_ref, pooled_ref)

    @pl.when(i == _NB - 1)
    def _():
        p = pooled_ref[...]
        po = jnp.maximum(jnp.dot(p, l1W_ref[...], precision=_HIGH,
                                 preferred_element_type=jnp.float32)
                         + l1b_ref[...], 0.0)
        out_ref[...] = (jnp.dot(po, l2W_ref[...], precision=_HIGH,
                                preferred_element_type=jnp.float32)
                        + l2b_ref[...])


def _mk_mlp(with_head):
    full = lambda s: pl.BlockSpec(s, lambda i: (0,) * len(s))
    row = pl.BlockSpec((_R, _H), lambda i: (i, 0))
    in_specs = [
        pl.BlockSpec(memory_space=pltpu.SMEM),
        pl.BlockSpec((1, 1, _R), lambda i: (i, 0, 0)),
        row,
        pl.BlockSpec((1, _R, _H), lambda i: (0, i, 0)),
        pl.BlockSpec((1, _R, _H), lambda i: (1, i, 0)),
        full((_H, _H)),
        full((1, _H)),
        full((1, _H)),
        full((1, _H)),
        full((_H, _H)),
        full((1, _H)),
    ]
    out_shapes = [
        jax.ShapeDtypeStruct((_N, _H), jnp.float32),
        jax.ShapeDtypeStruct((_G, _H), jnp.float32),
    ]
    out_specs = [row, full((_G, _H))]
    body = _mlp_body
    if with_head:
        in_specs += [full((_H, _H)), full((1, _H)), full((_H, _O)), full((1, _O))]
        out_shapes.append(jax.ShapeDtypeStruct((_G, _O), jnp.float32))
        out_specs.append(full((_G, _O)))
        body = _mlp_head_body
    return pl.pallas_call(
        body,
        grid=(_NB,),
        in_specs=in_specs,
        out_specs=out_specs,
        out_shape=out_shapes,
    )


_mlp_plain = _mk_mlp(False)
_mlp_head = _mk_mlp(True)


def kernel(x, edge_index, batch, W1_0, b1_0, gamma_0, beta_0, rm_0, rv_0, W2_0, b2_0, eps_0, W1_1, b1_1, gamma_1, beta_1, rm_1, rv_1, W2_1, b2_1, eps_1, W1_2, b1_2, gamma_2, beta_2, rm_2, rv_2, W2_2, b2_2, eps_2, lin1_W, lin1_b, lin2_W, lin2_b):
    p = dict(locals())
    src = edge_index[0].astype(jnp.int32)
    dst = edge_index[1].astype(jnp.int32)
    pad = _EPAD - _E
    pad_i = jnp.arange(pad, dtype=jnp.int32)
    src2d = jnp.reshape(
        jnp.concatenate([src, (pad_i * 997) % _N]), (_NCHUNK, _CH))
    dst2d = jnp.reshape(
        jnp.concatenate([dst, _N + (pad_i % _TRASH)]), (_NCHUNK, _CH))
    zeros = jnp.zeros((_RPT, _H), jnp.float32)
    batch3d = jnp.reshape(batch.astype(jnp.int32), (_NB, 1, _R))

    h = x
    pooled = []
    outp = None
    for i in range(_L):
        partials = _sc_agg(h, src2d, dst2d, zeros)
        seps = jnp.reshape(1.0 + p[f"eps_{i}"], (1, 1))
        scale = p[f"gamma_{i}"] / jnp.sqrt(p[f"rv_{i}"] + 1e-5)
        shift = p[f"beta_{i}"] - p[f"rm_{i}"] * scale
        args = (seps, batch3d, h, partials, partials,
                p[f"W1_{i}"], jnp.reshape(p[f"b1_{i}"], (1, _H)),
                jnp.reshape(scale, (1, _H)), jnp.reshape(shift, (1, _H)),
                p[f"W2_{i}"], jnp.reshape(p[f"b2_{i}"], (1, _H)))
        if i == _L - 1:
            h, pk, outp = _mlp_head(*args, lin1_W,
                                    jnp.reshape(lin1_b, (1, _H)),
                                    lin2_W, jnp.reshape(lin2_b, (1, _O)))
        else:
            h, pk = _mlp_plain(*args)
        pooled.append(pk)
    return (outp, *pooled)


# speedup vs baseline: 9.9936x; 1.0122x over previous
import functools

import jax
import jax.numpy as jnp
from jax import lax
from jax.experimental import pallas as pl
from jax.experimental.pallas import tpu as pltpu
from jax.experimental.pallas import tpu_sc as plsc

_N, _E, _D, _H, _O, _G, _L = 10000, 320000, 128, 128, 128, 64, 3

_NC, _NS = 2, 16
_NW = _NC * _NS
_CH = 128
_CPT = 80
_EPAD = _NW * _CPT * _CH
_NCHUNK = _EPAD // _CH
_TRASH = 112
_NACC = _N + _TRASH
_RPT = _NACC // _NS
_HCPT = _CPT // 2

_R = 1000
_NB = _N // _R

_HIGH = None


def _sc_agg_body(h_hbm, src_hbm, dst_hbm, zeros_hbm, out_hbm,
                 srcv, dstv, rows0, rows1, acc, g0, g1, s0, s1):
    c = lax.axis_index("c")
    s = lax.axis_index("s")
    tile = c * _NS + s
    r0 = s * _RPT

    pltpu.sync_copy(zeros_hbm, acc.at[pl.ds(r0, _RPT)])

    plsc.subcore_barrier()

    def step(jj, ra, ga, sa, rb, gb, sb):
        pltpu.make_async_copy(h_hbm.at[srcv.at[jj]], ra, ga).wait()
        pltpu.async_copy(ra, acc.at[dstv.at[jj]], sa, add=True)

        @pl.when(jj > 0)
        def _():
            pltpu.make_async_copy(rb, acc.at[dstv.at[jj - 1]], sb).wait()

        @pl.when(jj + 1 < _HCPT)
        def _():
            pltpu.async_copy(h_hbm.at[srcv.at[jj + 1]], rb, gb)

    for half in range(2):
        ch0 = tile * _CPT + half * _HCPT
        pltpu.sync_copy(src_hbm.at[pl.ds(ch0, _HCPT)], srcv)
        pltpu.sync_copy(dst_hbm.at[pl.ds(ch0, _HCPT)], dstv)
        pltpu.async_copy(h_hbm.at[srcv.at[0]], rows0, g0)

        def pair(k, carry):
            step(2 * k, rows0, g0, s0, rows1, g1, s1)
            step(2 * k + 1, rows1, g1, s1, rows0, g0, s0)
            return carry

        lax.fori_loop(0, _HCPT // 2, pair, 0)
        pltpu.make_async_copy(rows1, acc.at[dstv.at[_HCPT - 1]], s1).wait()

    plsc.subcore_barrier()
    pltpu.sync_copy(acc.at[pl.ds(r0, _RPT)], out_hbm.at[c, pl.ds(r0, _RPT)])


_sc_agg = functools.partial(
    pl.kernel,
    out_type=jax.ShapeDtypeStruct((_NC, _NACC, _H), jnp.float32),
    mesh=plsc.VectorSubcoreMesh(core_axis_name="c", subcore_axis_name="s",
                                num_cores=_NC, num_subcores=_NS),
    scratch_types=[
        pltpu.VMEM((_HCPT, _CH), jnp.int32),
        pltpu.VMEM((_HCPT, _CH), jnp.int32),
        pltpu.VMEM((_CH, _H), jnp.float32),
        pltpu.VMEM((_CH, _H), jnp.float32),
        pltpu.VMEM_SHARED((_NACC, _H), jnp.float32),
        pltpu.SemaphoreType.DMA,
        pltpu.SemaphoreType.DMA,
        pltpu.SemaphoreType.DMA,
        pltpu.SemaphoreType.DMA,
    ],
)(_sc_agg_body)


def _mlp_body_common(seps_ref, batch_ref, h_ref, p0_ref, p1_ref,
                     W1_ref, b1_ref, scale_ref, shift_ref, W2_ref, b2_ref,
                     hout_ref, pooled_ref):
    i = pl.program_id(0)
    h = h_ref[...]
    agg = p0_ref[0] + p1_ref[0]
    t = h * seps_ref[0, 0] + agg
    t = jnp.maximum(jnp.dot(t, W1_ref[...], precision=_HIGH,
                            preferred_element_type=jnp.float32) + b1_ref[...], 0.0)
    t = jnp.maximum(t * scale_ref[...] + shift_ref[...], 0.0)
    t = jnp.dot(t, W2_ref[...], precision=_HIGH,
                preferred_element_type=jnp.float32) + b2_ref[...]
    t = jnp.maximum(t, 0.0)
    hout_ref[...] = t

    bb = batch_ref[0, 0, :]
    seg = lax.broadcasted_iota(jnp.int32, (_G, _R), 0)
    oht = (seg == jnp.reshape(bb, (1, _R))).astype(jnp.float32)
    contrib = jnp.dot(oht, t, precision=_HIGH,
                      preferred_element_type=jnp.float32)

    @pl.when(i == 0)
    def _():
        pooled_ref[...] = contrib

    @pl.when(i != 0)
    def _():
        pooled_ref[...] += contrib

    return i


def _mlp_body(seps_ref, batch_ref, h_ref, p0_ref, p1_ref,
              W1_ref, b1_ref, scale_ref, shift_ref, W2_ref, b2_ref,
              hout_ref, pooled_ref):
    _mlp_body_common(seps_ref, batch_ref, h_ref, p0_ref, p1_ref,
                     W1_ref, b1_ref, scale_ref, shift_ref, W2_ref, b2_ref,
                     hout_ref, pooled_ref)


def _mlp_head_body(seps_ref, batch_ref, h_ref, p0_ref, p1_ref,
                   W1_ref, b1_ref, scale_ref, shift_ref, W2_ref, b2_ref,
                   l1W_ref, l1b_ref, l2W_ref, l2b_ref,
                   hout_ref, pooled_ref, out_ref):
    i = _mlp_body_common(seps_ref, batch_ref, h_ref, p0_ref, p1_ref,
                         W1_ref, b1_ref, scale_ref, shift_ref, W2_ref, b2_ref,
                         hout_ref, pooled_ref)

    @pl.when(i == _NB - 1)
    def _():
        p = pooled_ref[...]
        po = jnp.maximum(jnp.dot(p, l1W_ref[...], precision=_HIGH,
                                 preferred_element_type=jnp.float32)
                         + l1b_ref[...], 0.0)
        out_ref[...] = (jnp.dot(po, l2W_ref[...], precision=_HIGH,
                                preferred_element_type=jnp.float32)
                        + l2b_ref[...])


def _mk_mlp(with_head):
    full = lambda s: pl.BlockSpec(s, lambda i: (0,) * len(s))
    row = pl.BlockSpec((_R, _H), lambda i: (i, 0))
    in_specs = [
        pl.BlockSpec(memory_space=pltpu.SMEM),
        pl.BlockSpec((1, 1, _R), lambda i: (i, 0, 0)),
        row,
        pl.BlockSpec((1, _R, _H), lambda i: (0, i, 0)),
        pl.BlockSpec((1, _R, _H), lambda i: (1, i, 0)),
        full((_H, _H)),
        full((1, _H)),
        full((1, _H)),
        full((1, _H)),
        full((_H, _H)),
        full((1, _H)),
    ]
    out_shapes = [
        jax.ShapeDtypeStruct((_N, _H), jnp.float32),
        jax.ShapeDtypeStruct((_G, _H), jnp.float32),
    ]
    out_specs = [row, full((_G, _H))]
    body = _mlp_body
    if with_head:
        in_specs += [full((_H, _H)), full((1, _H)), full((_H, _O)), full((1, _O))]
        out_shapes.append(jax.ShapeDtypeStruct((_G, _O), jnp.float32))
        out_specs.append(full((_G, _O)))
        body = _mlp_head_body
    return pl.pallas_call(
        body,
        grid=(_NB,),
        in_specs=in_specs,
        out_specs=out_specs,
        out_shape=out_shapes,
    )


_mlp_plain = _mk_mlp(False)
_mlp_head = _mk_mlp(True)


def kernel(x, edge_index, batch, W1_0, b1_0, gamma_0, beta_0, rm_0, rv_0, W2_0, b2_0, eps_0, W1_1, b1_1, gamma_1, beta_1, rm_1, rv_1, W2_1, b2_1, eps_1, W1_2, b1_2, gamma_2, beta_2, rm_2, rv_2, W2_2, b2_2, eps_2, lin1_W, lin1_b, lin2_W, lin2_b):
    p = dict(locals())
    src = edge_index[0].astype(jnp.int32)
    dst = edge_index[1].astype(jnp.int32)
    pad = _EPAD - _E
    pad_i = jnp.arange(pad, dtype=jnp.int32)
    src2d = jnp.reshape(
        jnp.concatenate([src, (pad_i * 997) % _N]), (_NCHUNK, _CH))
    dst2d = jnp.reshape(
        jnp.concatenate([dst, _N + (pad_i % _TRASH)]), (_NCHUNK, _CH))
    zeros = jnp.zeros((_RPT, _H), jnp.float32)
    batch3d = jnp.reshape(batch.astype(jnp.int32), (_NB, 1, _R))

    h = x
    pooled = []
    outp = None
    for i in range(_L):
        partials = _sc_agg(h, src2d, dst2d, zeros)
        seps = jnp.reshape(1.0 + p[f"eps_{i}"], (1, 1))
        scale = p[f"gamma_{i}"] / jnp.sqrt(p[f"rv_{i}"] + 1e-5)
        shift = p[f"beta_{i}"] - p[f"rm_{i}"] * scale
        args = (seps, batch3d, h, partials, partials,
                p[f"W1_{i}"], jnp.reshape(p[f"b1_{i}"], (1, _H)),
                jnp.reshape(scale, (1, _H)), jnp.reshape(shift, (1, _H)),
                p[f"W2_{i}"], jnp.reshape(p[f"b2_{i}"], (1, _H)))
        if i == _L - 1:
            h, pk, outp = _mlp_head(*args, lin1_W,
                                    jnp.reshape(lin1_b, (1, _H)),
                                    lin2_W, jnp.reshape(lin2_b, (1, _O)))
        else:
            h, pk = _mlp_plain(*args)
        pooled.append(pk)
    return (outp, *pooled)


# speedup vs baseline: 10.0180x; 1.0024x over previous
import functools

import jax
import jax.numpy as jnp
from jax import lax
from jax.experimental import pallas as pl
from jax.experimental.pallas import tpu as pltpu
from jax.experimental.pallas import tpu_sc as plsc

_N, _E, _D, _H, _O, _G, _L = 10000, 320000, 128, 128, 128, 64, 3

_NC, _NS = 2, 16
_NW = _NC * _NS
_CH = 128
_CPT = 80
_EPAD = _NW * _CPT * _CH
_NCHUNK = _EPAD // _CH
_TRASH = 112
_NACC = _N + _TRASH
_RPT = _NACC // _NS
_HCPT = _CPT // 2

_R = 1000
_NB = _N // _R

_HIGH = None


def _sc_agg_body(h_hbm, src_hbm, dst_hbm, zeros_hbm, out_hbm,
                 srcv, dstv, rows0, rows1, acc, g0, g1, s0, s1):
    c = lax.axis_index("c")
    s = lax.axis_index("s")
    tile = c * _NS + s
    r0 = s * _RPT

    pltpu.sync_copy(zeros_hbm, acc.at[pl.ds(r0, _RPT)])

    plsc.subcore_barrier()

    def start_g(jj, rbuf, sem):
        pltpu.async_copy(h_hbm.at[srcv.at[jj, pl.ds(0, 64)]],
                         rbuf.at[pl.ds(0, 64)], sem)
        pltpu.async_copy(h_hbm.at[srcv.at[jj, pl.ds(64, 64)]],
                         rbuf.at[pl.ds(64, 64)], sem)

    def wait_g(jj, rbuf, sem):
        pltpu.make_async_copy(h_hbm.at[srcv.at[jj, pl.ds(0, 64)]],
                              rbuf.at[pl.ds(0, 64)], sem).wait()
        pltpu.make_async_copy(h_hbm.at[srcv.at[jj, pl.ds(64, 64)]],
                              rbuf.at[pl.ds(64, 64)], sem).wait()

    def step(jj, ra, ga, sa, rb, gb, sb):
        wait_g(jj, ra, ga)
        pltpu.async_copy(ra, acc.at[dstv.at[jj]], sa, add=True)

        @pl.when(jj > 0)
        def _():
            pltpu.make_async_copy(rb, acc.at[dstv.at[jj - 1]], sb).wait()

        @pl.when(jj + 1 < _HCPT)
        def _():
            start_g(jj + 1, rb, gb)

    for half in range(2):
        ch0 = tile * _CPT + half * _HCPT
        pltpu.sync_copy(src_hbm.at[pl.ds(ch0, _HCPT)], srcv)
        pltpu.sync_copy(dst_hbm.at[pl.ds(ch0, _HCPT)], dstv)
        start_g(0, rows0, g0)

        def pair(k, carry):
            step(2 * k, rows0, g0, s0, rows1, g1, s1)
            step(2 * k + 1, rows1, g1, s1, rows0, g0, s0)
            return carry

        lax.fori_loop(0, _HCPT // 2, pair, 0)
        pltpu.make_async_copy(rows1, acc.at[dstv.at[_HCPT - 1]], s1).wait()

    plsc.subcore_barrier()
    pltpu.sync_copy(acc.at[pl.ds(r0, _RPT)], out_hbm.at[c, pl.ds(r0, _RPT)])


_sc_agg = functools.partial(
    pl.kernel,
    out_type=jax.ShapeDtypeStruct((_NC, _NACC, _H), jnp.float32),
    mesh=plsc.VectorSubcoreMesh(core_axis_name="c", subcore_axis_name="s",
                                num_cores=_NC, num_subcores=_NS),
    scratch_types=[
        pltpu.VMEM((_HCPT, _CH), jnp.int32),
        pltpu.VMEM((_HCPT, _CH), jnp.int32),
        pltpu.VMEM((_CH, _H), jnp.float32),
        pltpu.VMEM((_CH, _H), jnp.float32),
        pltpu.VMEM_SHARED((_NACC, _H), jnp.float32),
        pltpu.SemaphoreType.DMA,
        pltpu.SemaphoreType.DMA,
        pltpu.SemaphoreType.DMA,
        pltpu.SemaphoreType.DMA,
    ],
)(_sc_agg_body)


def _mlp_body_common(seps_ref, batch_ref, h_ref, p0_ref, p1_ref,
                     W1_ref, b1_ref, scale_ref, shift_ref, W2_ref, b2_ref,
                     hout_ref, pooled_ref):
    i = pl.program_id(0)
    h = h_ref[...]
    agg = p0_ref[0] + p1_ref[0]
    t = h * seps_ref[0, 0] + agg
    t = jnp.maximum(jnp.dot(t, W1_ref[...], precision=_HIGH,
                            preferred_element_type=jnp.float32) + b1_ref[...], 0.0)
    t = jnp.maximum(t * scale_ref[...] + shift_ref[...], 0.0)
    t = jnp.dot(t, W2_ref[...], precision=_HIGH,
                preferred_element_type=jnp.float32) + b2_ref[...]
    t = jnp.maximum(t, 0.0)
    hout_ref[...] = t

    bb = batch_ref[0, 0, :]
    seg = lax.broadcasted_iota(jnp.int32, (_G, _R), 0)
    oht = (seg == jnp.reshape(bb, (1, _R))).astype(jnp.float32)
    contrib = jnp.dot(oht, t, precision=_HIGH,
                      preferred_element_type=jnp.float32)

    @pl.when(i == 0)
    def _():
        pooled_ref[...] = contrib

    @pl.when(i != 0)
    def _():
        pooled_ref[...] += contrib

    return i


def _mlp_body(seps_ref, batch_ref, h_ref, p0_ref, p1_ref,
              W1_ref, b1_ref, scale_ref, shift_ref, W2_ref, b2_ref,
              hout_ref, pooled_ref):
    _mlp_body_common(seps_ref, batch_ref, h_ref, p0_ref, p1_ref,
                     W1_ref, b1_ref, scale_ref, shift_ref, W2_ref, b2_ref,
                     hout_ref, pooled_ref)


def _mlp_head_body(seps_ref, batch_ref, h_ref, p0_ref, p1_ref,
                   W1_ref, b1_ref, scale_ref, shift_ref, W2_ref, b2_ref,
                   l1W_ref, l1b_ref, l2W_ref, l2b_ref,
                   hout_ref, pooled_ref, out_ref):
    i = _mlp_body_common(seps_ref, batch_ref, h_ref, p0_ref, p1_ref,
                         W1_ref, b1_ref, scale_ref, shift_ref, W2_ref, b2_ref,
                         hout_ref, pooled_ref)

    @pl.when(i == _NB - 1)
    def _():
        p = pooled_ref[...]
        po = jnp.maximum(jnp.dot(p, l1W_ref[...], precision=_HIGH,
                                 preferred_element_type=jnp.float32)
                         + l1b_ref[...], 0.0)
        out_ref[...] = (jnp.dot(po, l2W_ref[...], precision=_HIGH,
                                preferred_element_type=jnp.float32)
                        + l2b_ref[...])


def _mk_mlp(with_head):
    full = lambda s: pl.BlockSpec(s, lambda i: (0,) * len(s))
    row = pl.BlockSpec((_R, _H), lambda i: (i, 0))
    in_specs = [
        pl.BlockSpec(memory_space=pltpu.SMEM),
        pl.BlockSpec((1, 1, _R), lambda i: (i, 0, 0)),
        row,
        pl.BlockSpec((1, _R, _H), lambda i: (0, i, 0)),
        pl.BlockSpec((1, _R, _H), lambda i: (1, i, 0)),
        full((_H, _H)),
        full((1, _H)),
        full((1, _H)),
        full((1, _H)),
        full((_H, _H)),
        full((1, _H)),
    ]
    out_shapes = [
        jax.ShapeDtypeStruct((_N, _H), jnp.float32),
        jax.ShapeDtypeStruct((_G, _H), jnp.float32),
    ]
    out_specs = [row, full((_G, _H))]
    body = _mlp_body
    if with_head:
        in_specs += [full((_H, _H)), full((1, _H)), full((_H, _O)), full((1, _O))]
        out_shapes.append(jax.ShapeDtypeStruct((_G, _O), jnp.float32))
        out_specs.append(full((_G, _O)))
        body = _mlp_head_body
    return pl.pallas_call(
        body,
        grid=(_NB,),
        in_specs=in_specs,
        out_specs=out_specs,
        out_shape=out_shapes,
    )


_mlp_plain = _mk_mlp(False)
_mlp_head = _mk_mlp(True)


def kernel(x, edge_index, batch, W1_0, b1_0, gamma_0, beta_0, rm_0, rv_0, W2_0, b2_0, eps_0, W1_1, b1_1, gamma_1, beta_1, rm_1, rv_1, W2_1, b2_1, eps_1, W1_2, b1_2, gamma_2, beta_2, rm_2, rv_2, W2_2, b2_2, eps_2, lin1_W, lin1_b, lin2_W, lin2_b):
    p = dict(locals())
    src = edge_index[0].astype(jnp.int32)
    dst = edge_index[1].astype(jnp.int32)
    pad = _EPAD - _E
    pad_i = jnp.arange(pad, dtype=jnp.int32)
    src2d = jnp.reshape(
        jnp.concatenate([src, (pad_i * 997) % _N]), (_NCHUNK, _CH))
    dst2d = jnp.reshape(
        jnp.concatenate([dst, _N + (pad_i % _TRASH)]), (_NCHUNK, _CH))
    zeros = jnp.zeros((_RPT, _H), jnp.float32)
    batch3d = jnp.reshape(batch.astype(jnp.int32), (_NB, 1, _R))

    h = x
    pooled = []
    outp = None
    for i in range(_L):
        partials = _sc_agg(h, src2d, dst2d, zeros)
        seps = jnp.reshape(1.0 + p[f"eps_{i}"], (1, 1))
        scale = p[f"gamma_{i}"] / jnp.sqrt(p[f"rv_{i}"] + 1e-5)
        shift = p[f"beta_{i}"] - p[f"rm_{i}"] * scale
        args = (seps, batch3d, h, partials, partials,
                p[f"W1_{i}"], jnp.reshape(p[f"b1_{i}"], (1, _H)),
                jnp.reshape(scale, (1, _H)), jnp.reshape(shift, (1, _H)),
                p[f"W2_{i}"], jnp.reshape(p[f"b2_{i}"], (1, _H)))
        if i == _L - 1:
            h, pk, outp = _mlp_head(*args, lin1_W,
                                    jnp.reshape(lin1_b, (1, _H)),
                                    lin2_W, jnp.reshape(lin2_b, (1, _O)))
        else:
            h, pk = _mlp_plain(*args)
        pooled.append(pk)
    return (outp, *pooled)


# speedup vs baseline: 10.7437x; 1.0724x over previous
import functools

import jax
import jax.numpy as jnp
import numpy as np
from jax import lax
from jax.experimental import pallas as pl
from jax.experimental.pallas import tpu as pltpu
from jax.experimental.pallas import tpu_sc as plsc

_N, _E, _D, _H, _O, _G, _L = 10000, 320000, 128, 128, 128, 64, 3

_NC, _NS = 2, 16
_NW = _NC * _NS
_CH = 128
_CPT = 80
_EPAD = _NW * _CPT * _CH
_NCHUNK = _EPAD // _CH
_TRASH = 112
_NACC = _N + _TRASH
_RPT = _NACC // _NS
_HCPT = _CPT // 2

_R = 2000
_NB = _N // _R

_HIGH = None

_PADS = np.stack([
    (np.arange(_EPAD - _E) * 997) % _N,
    _N + (np.arange(_EPAD - _E) % _TRASH),
]).astype(np.int32)


def _sc_agg_body(h_hbm, e_hbm, out_hbm,
                 srcv, dstv, rows0, rows1, acc, g0, g1, s0, s1):
    c = lax.axis_index("c")
    s = lax.axis_index("s")
    tile = c * _NS + s
    r0 = s * _RPT

    def stage(half):
        ch0 = tile * _CPT + half * _HCPT
        pltpu.sync_copy(e_hbm.at[0, pl.ds(ch0, _HCPT)], srcv)
        pltpu.sync_copy(e_hbm.at[1, pl.ds(ch0, _HCPT)], dstv)

    def zfill(k, carry):
        for q in range(8):
            rows0[k, pl.ds(q * 16, 16)] = jnp.zeros((16,), jnp.float32)
        return carry

    lax.fori_loop(0, _CH, zfill, 0)
    for q in range(4):
        pltpu.sync_copy(rows0, acc.at[pl.ds(r0 + q * _CH, _CH)])
    pltpu.sync_copy(rows0.at[pl.ds(0, _RPT - 4 * _CH)],
                    acc.at[pl.ds(r0 + 4 * _CH, _RPT - 4 * _CH)])

    stage(0)
    plsc.subcore_barrier()

    def step(jj, ra, ga, sa, rb, gb, sb):
        pltpu.make_async_copy(h_hbm.at[srcv.at[jj]], ra, ga).wait()
        pltpu.async_copy(ra, acc.at[dstv.at[jj]], sa, add=True)

        @pl.when(jj > 0)
        def _():
            pltpu.make_async_copy(rb, acc.at[dstv.at[jj - 1]], sb).wait()

        @pl.when(jj + 1 < _HCPT)
        def _():
            pltpu.async_copy(h_hbm.at[srcv.at[jj + 1]], rb, gb)

    for half in range(2):
        if half:
            stage(half)
        pltpu.async_copy(h_hbm.at[srcv.at[0]], rows0, g0)

        def pair(k, carry):
            step(2 * k, rows0, g0, s0, rows1, g1, s1)
            step(2 * k + 1, rows1, g1, s1, rows0, g0, s0)
            return carry

        lax.fori_loop(0, _HCPT // 2, pair, 0)
        pltpu.make_async_copy(rows1, acc.at[dstv.at[_HCPT - 1]], s1).wait()

    plsc.subcore_barrier()
    pltpu.sync_copy(acc.at[pl.ds(r0, _RPT)], out_hbm.at[c, pl.ds(r0, _RPT)])


_sc_agg = functools.partial(
    pl.kernel,
    out_type=jax.ShapeDtypeStruct((_NC, _NACC, _H), jnp.float32),
    mesh=plsc.VectorSubcoreMesh(core_axis_name="c", subcore_axis_name="s",
                                num_cores=_NC, num_subcores=_NS),
    scratch_types=[
        pltpu.VMEM((_HCPT, _CH), jnp.int32),
        pltpu.VMEM((_HCPT, _CH), jnp.int32),
        pltpu.VMEM((_CH, _H), jnp.float32),
        pltpu.VMEM((_CH, _H), jnp.float32),
        pltpu.VMEM_SHARED((_NACC, _H), jnp.float32),
        pltpu.SemaphoreType.DMA,
        pltpu.SemaphoreType.DMA,
        pltpu.SemaphoreType.DMA,
        pltpu.SemaphoreType.DMA,
    ],
)(_sc_agg_body)


def _mlp_body_common(seps_ref, batch_ref, h_ref, p0_ref, p1_ref,
                     W1_ref, b1_ref, scale_ref, shift_ref, W2_ref, b2_ref,
                     hout_ref, pooled_ref):
    i = pl.program_id(0)
    h = h_ref[...]
    agg = p0_ref[0] + p1_ref[0]
    t = h * seps_ref[0, 0] + agg
    t = jnp.maximum(jnp.dot(t, W1_ref[...], precision=_HIGH,
                            preferred_element_type=jnp.float32) + b1_ref[...], 0.0)
    t = jnp.maximum(t * scale_ref[...] + shift_ref[...], 0.0)
    t = jnp.dot(t, W2_ref[...], precision=_HIGH,
                preferred_element_type=jnp.float32) + b2_ref[...]
    t = jnp.maximum(t, 0.0)
    hout_ref[...] = t

    bb = batch_ref[0, 0, :]
    seg = lax.broadcasted_iota(jnp.int32, (_G, _R), 0)
    oht = (seg == jnp.reshape(bb, (1, _R))).astype(jnp.float32)
    contrib = jnp.dot(oht, t, precision=_HIGH,
                      preferred_element_type=jnp.float32)

    @pl.when(i == 0)
    def _():
        pooled_ref[...] = contrib

    @pl.when(i != 0)
    def _():
        pooled_ref[...] += contrib

    return i


def _mlp_body(seps_ref, batch_ref, h_ref, p0_ref, p1_ref,
              W1_ref, b1_ref, scale_ref, shift_ref, W2_ref, b2_ref,
              hout_ref, pooled_ref):
    _mlp_body_common(seps_ref, batch_ref, h_ref, p0_ref, p1_ref,
                     W1_ref, b1_ref, scale_ref, shift_ref, W2_ref, b2_ref,
                     hout_ref, pooled_ref)


def _mlp_head_body(seps_ref, batch_ref, h_ref, p0_ref, p1_ref,
                   W1_ref, b1_ref, scale_ref, shift_ref, W2_ref, b2_ref,
                   l1W_ref, l1b_ref, l2W_ref, l2b_ref,
                   hout_ref, pooled_ref, out_ref):
    i = _mlp_body_common(seps_ref, batch_ref, h_ref, p0_ref, p1_ref,
                         W1_ref, b1_ref, scale_ref, shift_ref, W2_ref, b2_ref,
                         hout_ref, pooled_ref)

    @pl.when(i == _NB - 1)
    def _():
        p = pooled_ref[...]
        po = jnp.maximum(jnp.dot(p, l1W_ref[...], precision=_HIGH,
                                 preferred_element_type=jnp.float32)
                         + l1b_ref[...], 0.0)
        out_ref[...] = (jnp.dot(po, l2W_ref[...], precision=_HIGH,
                                preferred_element_type=jnp.float32)
                        + l2b_ref[...])


def _mk_mlp(with_head):
    full = lambda s: pl.BlockSpec(s, lambda i: (0,) * len(s))
    row = pl.BlockSpec((_R, _H), lambda i: (i, 0))
    in_specs = [
        pl.BlockSpec(memory_space=pltpu.SMEM),
        pl.BlockSpec((1, 1, _R), lambda i: (i, 0, 0)),
        row,
        pl.BlockSpec((1, _R, _H), lambda i: (0, i, 0)),
        pl.BlockSpec((1, _R, _H), lambda i: (1, i, 0)),
        full((_H, _H)),
        full((1, _H)),
        full((1, _H)),
        full((1, _H)),
        full((_H, _H)),
        full((1, _H)),
    ]
    out_shapes = [
        jax.ShapeDtypeStruct((_N, _H), jnp.float32),
        jax.ShapeDtypeStruct((_G, _H), jnp.float32),
    ]
    out_specs = [row, full((_G, _H))]
    body = _mlp_body
    if with_head:
        in_specs += [full((_H, _H)), full((1, _H)), full((_H, _O)), full((1, _O))]
        out_shapes.append(jax.ShapeDtypeStruct((_G, _O), jnp.float32))
        out_specs.append(full((_G, _O)))
        body = _mlp_head_body
    return pl.pallas_call(
        body,
        grid=(_NB,),
        in_specs=in_specs,
        out_specs=out_specs,
        out_shape=out_shapes,
    )


_mlp_plain = _mk_mlp(False)
_mlp_head = _mk_mlp(True)


def kernel(x, edge_index, batch, W1_0, b1_0, gamma_0, beta_0, rm_0, rv_0, W2_0, b2_0, eps_0, W1_1, b1_1, gamma_1, beta_1, rm_1, rv_1, W2_1, b2_1, eps_1, W1_2, b1_2, gamma_2, beta_2, rm_2, rv_2, W2_2, b2_2, eps_2, lin1_W, lin1_b, lin2_W, lin2_b):
    p = dict(locals())
    e3d = jnp.reshape(
        jnp.concatenate([edge_index.astype(jnp.int32), jnp.asarray(_PADS)], axis=1),
        (2, _NCHUNK, _CH))
    batch3d = jnp.reshape(batch.astype(jnp.int32), (_NB, 1, _R))

    h = x
    pooled = []
    outp = None
    for i in range(_L):
        partials = _sc_agg(h, e3d)
        seps = jnp.reshape(1.0 + p[f"eps_{i}"], (1, 1))
        scale = p[f"gamma_{i}"] / jnp.sqrt(p[f"rv_{i}"] + 1e-5)
        shift = p[f"beta_{i}"] - p[f"rm_{i}"] * scale
        args = (seps, batch3d, h, partials, partials,
                p[f"W1_{i}"], jnp.reshape(p[f"b1_{i}"], (1, _H)),
                jnp.reshape(scale, (1, _H)), jnp.reshape(shift, (1, _H)),
                p[f"W2_{i}"], jnp.reshape(p[f"b2_{i}"], (1, _H)))
        if i == _L - 1:
            h, pk, outp = _mlp_head(*args, lin1_W,
                                    jnp.reshape(lin1_b, (1, _H)),
                                    lin2_W, jnp.reshape(lin2_b, (1, _O)))
        else:
            h, pk = _mlp_plain(*args)
        pooled.append(pk)
    return (outp, *pooled)


# speedup vs baseline: 28.5725x; 2.6595x over previous
import functools

import jax
import jax.numpy as jnp
import numpy as np
from jax import lax
from jax.experimental import pallas as pl
from jax.experimental.pallas import tpu as pltpu
from jax.experimental.pallas import tpu_sc as plsc

_N, _E, _D, _H, _O, _G, _L = 10000, 320000, 128, 128, 128, 64, 3

_NC, _NS = 2, 16
_NW = _NC * _NS
_CH = 128
_CPT = 80
_EPAD = _NW * _CPT * _CH
_NCHUNK = _EPAD // _CH
_TRASH = 112
_NACC = _N + _TRASH
_RPT = _NACC // _NS
_HCPT = _CPT // 2

_R = 2000
_NB = _N // _R

_HIGH = None

_PADS = np.stack([
    (np.arange(_EPAD - _E) * 997) % _N,
    _N + (np.arange(_EPAD - _E) % _TRASH),
]).astype(np.int32)


def _sc_agg_body(h_hbm, e_hbm, out_hbm,
                 srcv, dstv, rows0, rows1, acc, g0, g1, s0, s1):
    c = lax.axis_index("c")
    s = lax.axis_index("s")
    tile = c * _NS + s
    r0 = s * _RPT

    def stage(half):
        ch0 = tile * _CPT + half * _HCPT
        pltpu.sync_copy(e_hbm.at[0, pl.ds(ch0, _HCPT)], srcv)
        pltpu.sync_copy(e_hbm.at[1, pl.ds(ch0, _HCPT)], dstv)

    def zfill(k, carry):
        for q in range(8):
            rows0[k, pl.ds(q * 16, 16)] = jnp.zeros((16,), jnp.float32)
        return carry

    lax.fori_loop(0, _CH, zfill, 0)
    for q in range(4):
        pltpu.sync_copy(rows0, acc.at[pl.ds(r0 + q * _CH, _CH)])
    pltpu.sync_copy(rows0.at[pl.ds(0, _RPT - 4 * _CH)],
                    acc.at[pl.ds(r0 + 4 * _CH, _RPT - 4 * _CH)])

    stage(0)
    plsc.subcore_barrier()

    def step(jj, ra, ga, sa, rb, gb, sb):
        pltpu.make_async_copy(h_hbm.at[srcv.at[jj]], ra, ga).wait()
        pltpu.async_copy(ra, acc.at[dstv.at[jj]], sa, add=True)

        @pl.when(jj > 0)
        def _():
            pltpu.make_async_copy(rb, acc.at[dstv.at[jj - 1]], sb).wait()

        @pl.when(jj + 1 < _HCPT)
        def _():
            pltpu.async_copy(h_hbm.at[srcv.at[jj + 1]], rb, gb)

    for half in range(2):
        if half:
            stage(half)
        pltpu.async_copy(h_hbm.at[srcv.at[0]], rows0, g0)

        def pair(k, carry):
            step(2 * k, rows0, g0, s0, rows1, g1, s1)
            step(2 * k + 1, rows1, g1, s1, rows0, g0, s0)
            return carry

        lax.fori_loop(0, _HCPT // 2, pair, 0)
        pltpu.make_async_copy(rows1, acc.at[dstv.at[_HCPT - 1]], s1).wait()

    plsc.subcore_barrier()
    pltpu.sync_copy(acc.at[pl.ds(r0, _RPT)], out_hbm.at[c, pl.ds(r0, _RPT)])


_sc_agg = functools.partial(
    pl.kernel,
    out_type=jax.ShapeDtypeStruct((_NC, _NACC, _H), jnp.float32),
    mesh=plsc.VectorSubcoreMesh(core_axis_name="c", subcore_axis_name="s",
                                num_cores=_NC, num_subcores=_NS),
    scratch_types=[
        pltpu.VMEM((_HCPT, _CH), jnp.int32),
        pltpu.VMEM((_HCPT, _CH), jnp.int32),
        pltpu.VMEM((_CH, _H), jnp.float32),
        pltpu.VMEM((_CH, _H), jnp.float32),
        pltpu.VMEM_SHARED((_NACC, _H), jnp.float32),
        pltpu.SemaphoreType.DMA,
        pltpu.SemaphoreType.DMA,
        pltpu.SemaphoreType.DMA,
        pltpu.SemaphoreType.DMA,
    ],
)(_sc_agg_body)


def _mlp_body(seps_ref, h_ref, p0_ref, p1_ref,
              W1_ref, b1_ref, scale_ref, shift_ref, W2_ref, b2_ref,
              hout_ref):
    h = h_ref[...]
    agg = p0_ref[0] + p1_ref[0]
    t = h * seps_ref[0, 0] + agg
    t = jnp.maximum(jnp.dot(t, W1_ref[...], precision=_HIGH,
                            preferred_element_type=jnp.float32) + b1_ref[...], 0.0)
    t = jnp.maximum(t * scale_ref[...] + shift_ref[...], 0.0)
    t = jnp.dot(t, W2_ref[...], precision=_HIGH,
                preferred_element_type=jnp.float32) + b2_ref[...]
    hout_ref[...] = jnp.maximum(t, 0.0)


def _pool_body_common(batch_ref, h_ref, pooled_ref):
    i = pl.program_id(0)
    bb = batch_ref[0, 0, :]
    seg = lax.broadcasted_iota(jnp.int32, (_G, _R), 0)
    oht = (seg == jnp.reshape(bb, (1, _R))).astype(jnp.float32)
    contrib = jnp.dot(oht, h_ref[...], precision=_HIGH,
                      preferred_element_type=jnp.float32)

    @pl.when(i == 0)
    def _():
        pooled_ref[...] = contrib

    @pl.when(i != 0)
    def _():
        pooled_ref[...] += contrib

    return i


def _pool_body(batch_ref, h_ref, pooled_ref):
    _pool_body_common(batch_ref, h_ref, pooled_ref)


def _pool_head_body(batch_ref, h_ref, l1W_ref, l1b_ref, l2W_ref, l2b_ref,
                    pooled_ref, out_ref):
    i = _pool_body_common(batch_ref, h_ref, pooled_ref)

    @pl.when(i == _NB - 1)
    def _():
        p = pooled_ref[...]
        po = jnp.maximum(jnp.dot(p, l1W_ref[...], precision=_HIGH,
                                 preferred_element_type=jnp.float32)
                         + l1b_ref[...], 0.0)
        out_ref[...] = (jnp.dot(po, l2W_ref[...], precision=_HIGH,
                                preferred_element_type=jnp.float32)
                        + l2b_ref[...])


def _mk_full(shape):
    return pl.BlockSpec(shape, lambda i: (0,) * len(shape))


_ROWSPEC = pl.BlockSpec((_R, _H), lambda i: (i, 0))

_mlp = pl.pallas_call(
    _mlp_body,
    grid=(_NB,),
    in_specs=[
        pl.BlockSpec(memory_space=pltpu.SMEM),
        _ROWSPEC,
        pl.BlockSpec((1, _R, _H), lambda i: (0, i, 0)),
        pl.BlockSpec((1, _R, _H), lambda i: (1, i, 0)),
        _mk_full((_H, _H)),
        _mk_full((1, _H)),
        _mk_full((1, _H)),
        _mk_full((1, _H)),
        _mk_full((_H, _H)),
        _mk_full((1, _H)),
    ],
    out_specs=_ROWSPEC,
    out_shape=jax.ShapeDtypeStruct((_N, _H), jnp.float32),
)

_BATCHSPEC = pl.BlockSpec((1, 1, _R), lambda i: (i, 0, 0))

_pool_plain = pl.pallas_call(
    _pool_body,
    grid=(_NB,),
    in_specs=[_BATCHSPEC, _ROWSPEC],
    out_specs=_mk_full((_G, _H)),
    out_shape=jax.ShapeDtypeStruct((_G, _H), jnp.float32),
)

_pool_head = pl.pallas_call(
    _pool_head_body,
    grid=(_NB,),
    in_specs=[_BATCHSPEC, _ROWSPEC, _mk_full((_H, _H)), _mk_full((1, _H)),
              _mk_full((_H, _O)), _mk_full((1, _O))],
    out_specs=[_mk_full((_G, _H)), _mk_full((_G, _O))],
    out_shape=[jax.ShapeDtypeStruct((_G, _H), jnp.float32),
               jax.ShapeDtypeStruct((_G, _O), jnp.float32)],
)


def kernel(x, edge_index, batch, W1_0, b1_0, gamma_0, beta_0, rm_0, rv_0, W2_0, b2_0, eps_0, W1_1, b1_1, gamma_1, beta_1, rm_1, rv_1, W2_1, b2_1, eps_1, W1_2, b1_2, gamma_2, beta_2, rm_2, rv_2, W2_2, b2_2, eps_2, lin1_W, lin1_b, lin2_W, lin2_b):
    p = dict(locals())
    e3d = jnp.reshape(
        jnp.concatenate([edge_index.astype(jnp.int32), jnp.asarray(_PADS)], axis=1),
        (2, _NCHUNK, _CH))
    batch3d = jnp.reshape(batch.astype(jnp.int32), (_NB, 1, _R))

    h = x
    pooled = []
    outp = None
    for i in range(_L):
        partials = _sc_agg(h, e3d)
        seps = jnp.reshape(1.0 + p[f"eps_{i}"], (1, 1))
        scale = p[f"gamma_{i}"] / jnp.sqrt(p[f"rv_{i}"] + 1e-5)
        shift = p[f"beta_{i}"] - p[f"rm_{i}"] * scale
        h = _mlp(seps, h, partials, partials,
                 p[f"W1_{i}"], jnp.reshape(p[f"b1_{i}"], (1, _H)),
                 jnp.reshape(scale, (1, _H)), jnp.reshape(shift, (1, _H)),
                 p[f"W2_{i}"], jnp.reshape(p[f"b2_{i}"], (1, _H)))
        if i == _L - 1:
            pk, outp = _pool_head(batch3d, h, lin1_W,
                                  jnp.reshape(lin1_b, (1, _H)),
                                  lin2_W, jnp.reshape(lin2_b, (1, _O)))
        else:
            pk = _pool_plain(batch3d, h)
        pooled.append(pk)
        return (outp, *pooled)
